# baseline (speedup 1.0000x reference)
import functools

import jax
import jax.numpy as jnp
from jax import lax
from jax.experimental import pallas as pl
from jax.experimental.pallas import tpu as pltpu
from jax.experimental.pallas import tpu_sc as plsc

BATCH = 16384
FIELDS = 26
DIM = 64
NUM_CORES = 2
NUM_SUBCORES = 16
NW = NUM_CORES * NUM_SUBCORES
B_PER_W = BATCH // NW
CHUNK = 128
KBLK = B_PER_W // CHUNK
CHUNKS = FIELDS * KBLK
NBUF = 4
LOOKAHEAD = 2
LANES = 16
SUB = 8


def _make_kernel():
    mesh = plsc.VectorSubcoreMesh(core_axis_name="c", subcore_axis_name="s")

    @functools.partial(
        pl.kernel,
        mesh=mesh,
        out_type=jax.ShapeDtypeStruct(
            (FIELDS, SUB, BATCH // CHUNK, SUB, CHUNK), jnp.float32),
        scratch_types=(
            [pltpu.VMEM((FIELDS, B_PER_W), jnp.int32)]
            + [pltpu.VMEM((CHUNK, DIM), jnp.float32) for _ in range(NBUF)]
            + [pltpu.VMEM((SUB, SUB, CHUNK), jnp.float32) for _ in range(NBUF)]
            + [pltpu.SemaphoreType.DMA((NBUF,)), pltpu.SemaphoreType.DMA((NBUF,))]
        ),
        compiler_params=pltpu.CompilerParams(
            use_tc_tiling_on_sc=False, needs_layout_passes=False),
    )
    def body(xt_hbm, w_hbm, out_hbm, idx_v, *rest):
        rows = rest[:NBUF]
        tbuf = rest[NBUF:2 * NBUF]
        gsem, wsem = rest[2 * NBUF], rest[2 * NBUF + 1]
        wid = lax.axis_index("s") * NUM_CORES + lax.axis_index("c")
        base_bt = wid * KBLK
        pltpu.sync_copy(
            xt_hbm.at[:, pl.ds(base_bt * CHUNK, B_PER_W)], idx_v)

        def fire_gather(c, b):
            f = c // KBLK
            k = c % KBLK
            pltpu.async_copy(
                w_hbm.at[idx_v.at[f, pl.ds(k * CHUNK, CHUNK)]], rows[b],
                gsem.at[b])

        def wait_gather(b):
            pltpu.make_async_copy(
                w_hbm.at[idx_v.at[0, pl.ds(0, CHUNK)]], rows[b],
                gsem.at[b]).wait()

        def transpose(b):
            def col(d, carry):
                dt = lax.div(d, SUB)
                s = lax.rem(d, SUB)
                dvec = jnp.full((LANES,), d, dtype=jnp.int32)
                for k in range(CHUNK // LANES):
                    lvec = jax.lax.iota(jnp.int32, LANES) + k * LANES
                    v = plsc.load_gather(rows[b], [lvec, dvec])
                    tbuf[b][dt, s, pl.ds(k * LANES, LANES)] = v
                return carry

            lax.fori_loop(0, DIM, col, 0)

        def fire_write(c, b):
            f = c // KBLK
            k = c % KBLK
            pltpu.async_copy(
                tbuf[b],
                out_hbm.at[f, pl.ds(0, SUB), base_bt + k],
                wsem.at[b])

        def wait_write(b):
            pltpu.make_async_copy(
                tbuf[b], out_hbm.at[0, pl.ds(0, SUB), 0], wsem.at[b]).wait()

        for b in range(LOOKAHEAD):
            fire_gather(b, b)

        for b in range(NBUF):
            wait_gather(b)
            fire_gather(b + LOOKAHEAD, (b + LOOKAHEAD) % NBUF)
            transpose(b)
            fire_write(b, b)

        def block(gi, carry):
            g = gi * NBUF
            for b in range(NBUF):
                c = g + b
                wait_gather(b)
                bb = (b + LOOKAHEAD) % NBUF
                fire_gather(c + LOOKAHEAD, bb)
                wait_write(b)
                transpose(b)
                fire_write(c, b)
            return carry

        lax.fori_loop(1, CHUNKS // NBUF - 1, block, 0)

        g = CHUNKS - NBUF
        for b in range(NBUF):
            c = g + b
            wait_gather(b)
            if b < LOOKAHEAD:
                bb = (b + LOOKAHEAD) % NBUF
                fire_gather(c + LOOKAHEAD, bb)
            wait_write(b)
            transpose(b)
            fire_write(c, b)

        for b in range(NBUF):
            wait_write(b)

    return body


_kern = _make_kernel()


def kernel(x, weight):
    xt = x.T.astype(jnp.int32)
    out5 = _kern(xt, weight)
    return out5.transpose(2, 4, 0, 1, 3).reshape(BATCH, FIELDS, DIM)
